# baseline (speedup 1.0000x reference)
import functools

import jax
import jax.numpy as jnp
from jax import lax
from jax.experimental import pallas as pl
from jax.experimental.pallas import tpu as pltpu
from jax.experimental.pallas import tpu_sc as plsc

_B = 16
_NQ = 5000
_NQP = 5120
_NC = 80
_K = 300
_KP = 384
_L = 16
_NCHUNK = _NQP // _L


def _dense_kernel(logits_ref, bits_out, labels_out, ustar_out, need_out):
    i = pl.program_id(0)
    x = logits_ref[0]
    p = 1.0 / (1.0 + jnp.exp(-x))
    s = jnp.max(p, axis=1)
    cls_iota = lax.broadcasted_iota(jnp.int32, (_NQ, _NC), 1)
    lab = jnp.min(jnp.where(p == s[:, None], cls_iota, _NC), axis=1)
    sbits = lax.bitcast_convert_type(s, jnp.int32)
    zpad = jnp.zeros((_NQP - _NQ,), jnp.int32)
    bits_out[pl.ds(i, 1), :] = jnp.concatenate([sbits, zpad])[None]
    labels_out[pl.ds(i, 1), :] = jnp.concatenate([lab, zpad])[None]

    @pl.when(i == _B - 1)
    def _threshold():
        bits_all = bits_out[...]

        def bs_body(_, lohi):
            lo, hi = lohi
            mid = lo + ((hi - lo) >> 1)
            cnt = jnp.sum((bits_all > mid).astype(jnp.int32),
                          axis=1, keepdims=True)
            less = cnt < _K
            return jnp.where(less, lo, mid), jnp.where(less, mid, hi)

        lo0 = jnp.zeros((_B, 1), jnp.int32)
        hi0 = jnp.full((_B, 1), 1 << 30, jnp.int32)
        _, ustar = lax.fori_loop(0, 30, bs_body, (lo0, hi0))
        m = jnp.sum((bits_all > ustar).astype(jnp.int32),
                    axis=1, keepdims=True)
        need = _K - m
        ustar_out[...] = jnp.broadcast_to(ustar, (_B, 128))
        need_out[...] = jnp.broadcast_to(need, (_B, 128))


def _stage_a(logits):
    return pl.pallas_call(
        _dense_kernel,
        grid=(_B,),
        in_specs=[pl.BlockSpec((1, _NQ, _NC), lambda i: (i, 0, 0))],
        out_specs=[
            pl.BlockSpec((_B, _NQP), lambda i: (0, 0)),
            pl.BlockSpec((_B, _NQP), lambda i: (0, 0)),
            pl.BlockSpec((_B, 128), lambda i: (0, 0)),
            pl.BlockSpec((_B, 128), lambda i: (0, 0)),
        ],
        out_shape=[
            jax.ShapeDtypeStruct((_B, _NQP), jnp.int32),
            jax.ShapeDtypeStruct((_B, _NQP), jnp.int32),
            jax.ShapeDtypeStruct((_B, 128), jnp.int32),
            jax.ShapeDtypeStruct((_B, 128), jnp.int32),
        ],
    )(logits)


def _sc_body(bits_hbm, lab_hbm, cx_hbm, cy_hbm, w_hbm, h_hbm,
             ustar_hbm, need_hbm, base_hbm,
             obits_hbm, oidx_hbm, olab_hbm, ocx_hbm, ocy_hbm, ow_hbm, oh_hbm,
             bits_v, ustar_v, need_v, base_v,
             obits_v, oidx_v, gidx_v, olab_v, ocx_v, ocy_v, ow_v, oh_v, sem):
    wid = lax.axis_index("s") * 2 + lax.axis_index("c")

    @pl.when(wid < _B)
    def _work():
        img = wid
        pltpu.sync_copy(bits_hbm.at[img], bits_v)
        pltpu.sync_copy(ustar_hbm.at[img], ustar_v)
        pltpu.sync_copy(need_hbm.at[img], need_v)
        pltpu.sync_copy(base_hbm.at[img], base_v)
        iota16 = lax.broadcasted_iota(jnp.int32, (_L,), 0)
        zero16 = iota16 * 0
        ustar = ustar_v[pl.ds(0, _L)]
        need = need_v[pl.ds(0, _L)]
        base = base_v[pl.ds(0, _L)]

        for t in range(288, _KP, _L):
            obits_v[pl.ds(t, _L)] = zero16 - 1
            oidx_v[pl.ds(t, _L)] = zero16

        def chunk(c, carry):
            c_sel, c_eq, idxv = carry
            v = bits_v[pl.ds(c * _L, _L)]
            gt = v > ustar
            eq = v == ustar
            eqi = eq.astype(jnp.int32)
            excl_eq = plsc.cumsum(eqi) - eqi
            sel = gt | (eq & ((c_eq + excl_eq) < need))
            plsc.store_compressed(obits_v.at[pl.ds(c_sel, _L)], v, mask=sel)
            plsc.store_compressed(oidx_v.at[pl.ds(c_sel, _L)], idxv, mask=sel)
            return (c_sel + jnp.sum(sel.astype(jnp.int32)),
                    c_eq + plsc.all_reduce_population_count(eq),
                    idxv + _L)

        lax.fori_loop(0, _NCHUNK, chunk,
                      (jnp.int32(0), zero16, iota16))

        for g in range(_KP // _L):
            gidx_v[pl.ds(g * _L, _L)] = oidx_v[pl.ds(g * _L, _L)] + base

        pltpu.async_copy(lab_hbm.at[gidx_v], olab_v, sem).wait()
        pltpu.async_copy(cx_hbm.at[gidx_v], ocx_v, sem).wait()
        pltpu.async_copy(cy_hbm.at[gidx_v], ocy_v, sem).wait()
        pltpu.async_copy(w_hbm.at[gidx_v], ow_v, sem).wait()
        pltpu.async_copy(h_hbm.at[gidx_v], oh_v, sem).wait()

        pltpu.sync_copy(obits_v.at[pl.ds(0, _KP)], obits_hbm.at[img])
        pltpu.sync_copy(oidx_v.at[pl.ds(0, _KP)], oidx_hbm.at[img])
        pltpu.sync_copy(olab_v, olab_hbm.at[img])
        pltpu.sync_copy(ocx_v, ocx_hbm.at[img])
        pltpu.sync_copy(ocy_v, ocy_hbm.at[img])
        pltpu.sync_copy(ow_v, ow_hbm.at[img])
        pltpu.sync_copy(oh_v, oh_hbm.at[img])


def _sc_compact_gather(bits, labels, cxp, cyp, wp, hp, meta):
    mesh = plsc.VectorSubcoreMesh(core_axis_name="c", subcore_axis_name="s")
    f32, i32 = jnp.float32, jnp.int32
    run = functools.partial(
        pl.kernel, mesh=mesh,
        compiler_params=pltpu.CompilerParams(needs_layout_passes=False),
        out_type=[
            jax.ShapeDtypeStruct((_B, _KP), i32),
            jax.ShapeDtypeStruct((_B, _KP), i32),
            jax.ShapeDtypeStruct((_B, _KP), i32),
            jax.ShapeDtypeStruct((_B, _KP), f32),
            jax.ShapeDtypeStruct((_B, _KP), f32),
            jax.ShapeDtypeStruct((_B, _KP), f32),
            jax.ShapeDtypeStruct((_B, _KP), f32),
        ],
        scratch_types=[
            pltpu.VMEM((_NQP,), i32),
            pltpu.VMEM((128,), i32),
            pltpu.VMEM((128,), i32),
            pltpu.VMEM((128,), i32),
            pltpu.VMEM((_KP + _L,), i32),
            pltpu.VMEM((_KP + _L,), i32),
            pltpu.VMEM((_KP,), i32),
            pltpu.VMEM((_KP,), i32),
            pltpu.VMEM((_KP,), f32),
            pltpu.VMEM((_KP,), f32),
            pltpu.VMEM((_KP,), f32),
            pltpu.VMEM((_KP,), f32),
            pltpu.SemaphoreType.DMA,
        ],
    )(_sc_body)
    base = jnp.broadcast_to(
        (jnp.arange(_B, dtype=i32) * _NQP)[:, None], (_B, 128))
    return run(bits, labels.reshape(-1), cxp.reshape(-1), cyp.reshape(-1),
               wp.reshape(-1), hp.reshape(-1),
               meta[0], meta[1], base)


def _order_kernel(bits_row_ref, bits_col_ref, idx_row_ref, idx_col_ref,
                  lab_ref, cx_ref, cy_ref, w_ref, h_ref, sizes_ref,
                  scores_out, labels_out, x1_out, y1_out, x2_out, y2_out):
    brow = bits_row_ref[0]
    bcol = bits_col_ref[0]
    irow = idx_row_ref[0]
    icol = idx_col_ref[0]

    before = (brow > bcol) | ((brow == bcol) & (irow < icol))
    rank = jnp.sum(before.astype(jnp.int32), axis=1, keepdims=True)

    outpos = lax.broadcasted_iota(jnp.int32, (_KP, _KP), 1)
    perm = (rank == outpos).astype(jnp.float32)

    srow = jnp.where(brow >= 0,
                     lax.bitcast_convert_type(brow, jnp.float32), 0.0)
    lrow = lab_ref[0].astype(jnp.float32)
    stream_i = lax.broadcasted_iota(jnp.int32, (8, _KP), 0)
    vt = jnp.where(stream_i == 0, jnp.broadcast_to(srow, (8, _KP)), 0.0)
    vt = jnp.where(stream_i == 1, jnp.broadcast_to(lrow, (8, _KP)), vt)
    vt = jnp.where(stream_i == 2, jnp.broadcast_to(cx_ref[0], (8, _KP)), vt)
    vt = jnp.where(stream_i == 3, jnp.broadcast_to(cy_ref[0], (8, _KP)), vt)
    vt = jnp.where(stream_i == 4, jnp.broadcast_to(w_ref[0], (8, _KP)), vt)
    vt = jnp.where(stream_i == 5, jnp.broadcast_to(h_ref[0], (8, _KP)), vt)

    r = lax.dot_general(vt, perm, (((1,), (0,)), ((), ())),
                        precision=lax.Precision.HIGHEST,
                        preferred_element_type=jnp.float32)

    i = pl.program_id(0)
    hrow = sizes_ref[pl.ds(i, 1), 0:1]
    wrow = sizes_ref[pl.ds(i, 1), 1:2]
    img_h = jnp.broadcast_to(hrow, (1, _KP))
    img_w = jnp.broadcast_to(wrow, (1, _KP))
    cxa = r[2:3, :]
    cya = r[3:4, :]
    wa = r[4:5, :]
    ha = r[5:6, :]
    scores_out[0] = r[0:1, :]
    labels_out[0] = r[1:2, :].astype(jnp.int32)
    x1_out[0] = (cxa - 0.5 * wa) * img_w
    y1_out[0] = (cya - 0.5 * ha) * img_h
    x2_out[0] = (cxa + 0.5 * wa) * img_w
    y2_out[0] = (cya + 0.5 * ha) * img_h


def _stage_c(cbits, cidx, clab, ccx, ccy, cw, ch, target_sizes):
    row = lambda a: a.reshape(_B, 1, _KP)
    col = lambda a: a.reshape(_B, _KP, 1)
    rspec = pl.BlockSpec((1, 1, _KP), lambda i: (i, 0, 0))
    cspec = pl.BlockSpec((1, _KP, 1), lambda i: (i, 0, 0))
    outs = pl.pallas_call(
        _order_kernel,
        grid=(_B,),
        in_specs=[rspec, cspec, rspec, cspec,
                  rspec, rspec, rspec, rspec, rspec,
                  pl.BlockSpec((_B, 2), lambda i: (0, 0))],
        out_specs=[rspec] * 6,
        out_shape=[
            jax.ShapeDtypeStruct((_B, 1, _KP), jnp.float32),
            jax.ShapeDtypeStruct((_B, 1, _KP), jnp.int32),
            jax.ShapeDtypeStruct((_B, 1, _KP), jnp.float32),
            jax.ShapeDtypeStruct((_B, 1, _KP), jnp.float32),
            jax.ShapeDtypeStruct((_B, 1, _KP), jnp.float32),
            jax.ShapeDtypeStruct((_B, 1, _KP), jnp.float32),
        ],
    )(row(cbits), col(cbits), row(cidx), col(cidx),
      row(clab), row(ccx), row(ccy), row(cw), row(ch), target_sizes)
    return [o.reshape(_B, _KP)[:, :_K] for o in outs]


def kernel(pred_logits, pred_boxes, target_sizes):
    pad_q = lambda a: jnp.pad(a, ((0, 0), (0, _NQP - _NQ)))
    cxp = pad_q(pred_boxes[..., 0])
    cyp = pad_q(pred_boxes[..., 1])
    wp = pad_q(pred_boxes[..., 2])
    hp = pad_q(pred_boxes[..., 3])

    bits, labels, ustar_arr, need_arr = _stage_a(pred_logits)
    comp = _sc_compact_gather(bits, labels, cxp, cyp, wp, hp,
                              (ustar_arr, need_arr))
    top_scores, top_labels, x1, y1, x2, y2 = _stage_c(*comp, target_sizes)
    top_boxes = jnp.stack([x1, y1, x2, y2], axis=-1)
    return top_scores, top_labels, top_boxes


# speedup vs baseline: 2.0105x; 2.0105x over previous
import functools

import jax
import jax.numpy as jnp
from jax import lax
from jax.experimental import pallas as pl
from jax.experimental.pallas import tpu as pltpu
from jax.experimental.pallas import tpu_sc as plsc

_B = 16
_NQ = 5000
_NQP = 5120
_NC = 80
_K = 300
_KP = 384
_L = 16
_NCHUNK = _NQP // _L


def _dense_kernel(logits_t_ref, bits_out, labels_out, ustar_out, need_out):
    i = pl.program_id(0)
    x = logits_t_ref[0]
    p = 1.0 / (1.0 + jnp.exp(-x))
    s = jnp.max(p, axis=0)
    cls_iota = lax.broadcasted_iota(jnp.int32, (_NC, _NQ), 0)
    lab = jnp.min(jnp.where(p == s[None, :], cls_iota, _NC), axis=0)
    zpad = jnp.zeros((_NQP - _NQ,), jnp.int32)
    bits_out[pl.ds(i, 1), :] = jnp.concatenate(
        [lax.bitcast_convert_type(s, jnp.int32), zpad])[None]
    labels_out[pl.ds(i, 1), :] = jnp.concatenate([lab, zpad])[None]

    @pl.when(i == _B - 1)
    def _threshold():
        bits_all = bits_out[...]

        def bs_body(_, lohi):
            lo, hi = lohi
            mid = lo + ((hi - lo) >> 1)
            cnt = jnp.sum((bits_all > mid).astype(jnp.int32),
                          axis=1, keepdims=True)
            less = cnt < _K
            return jnp.where(less, lo, mid), jnp.where(less, mid, hi)

        lo0 = jnp.zeros((_B, 1), jnp.int32)
        hi0 = jnp.full((_B, 1), 1 << 30, jnp.int32)
        _, ustar = lax.fori_loop(0, 30, bs_body, (lo0, hi0))
        m = jnp.sum((bits_all > ustar).astype(jnp.int32),
                    axis=1, keepdims=True)
        need = _K - m
        ustar_out[...] = jnp.broadcast_to(ustar, (_B, 128))
        need_out[...] = jnp.broadcast_to(need, (_B, 128))


def _stage_a(logits_tp):
    return pl.pallas_call(
        _dense_kernel,
        grid=(_B,),
        in_specs=[pl.BlockSpec((1, _NC, _NQ), lambda i: (i, 0, 0))],
        out_specs=[
            pl.BlockSpec((_B, _NQP), lambda i: (0, 0)),
            pl.BlockSpec((_B, _NQP), lambda i: (0, 0)),
            pl.BlockSpec((_B, 128), lambda i: (0, 0)),
            pl.BlockSpec((_B, 128), lambda i: (0, 0)),
        ],
        out_shape=[
            jax.ShapeDtypeStruct((_B, _NQP), jnp.int32),
            jax.ShapeDtypeStruct((_B, _NQP), jnp.int32),
            jax.ShapeDtypeStruct((_B, 128), jnp.int32),
            jax.ShapeDtypeStruct((_B, 128), jnp.int32),
        ],
    )(logits_tp)


def _sc_body(bits_hbm, lab_hbm, cx_hbm, cy_hbm, w_hbm, h_hbm,
             ustar_hbm, need_hbm, base_hbm,
             obits_hbm, oidx_hbm, olab_hbm, ocx_hbm, ocy_hbm, ow_hbm, oh_hbm,
             bits_v, ustar_v, need_v, base_v,
             obits_v, oidx_v, gidx_v, olab_v, ocx_v, ocy_v, ow_v, oh_v, sem):
    wid = lax.axis_index("s") * 2 + lax.axis_index("c")

    @pl.when(wid < _B)
    def _work():
        img = wid
        pltpu.sync_copy(bits_hbm.at[img], bits_v)
        pltpu.sync_copy(ustar_hbm.at[img], ustar_v)
        pltpu.sync_copy(need_hbm.at[img], need_v)
        pltpu.sync_copy(base_hbm.at[img], base_v)
        iota16 = lax.broadcasted_iota(jnp.int32, (_L,), 0)
        zero16 = iota16 * 0
        ustar = ustar_v[pl.ds(0, _L)]
        need = need_v[pl.ds(0, _L)]
        base = base_v[pl.ds(0, _L)]

        for t in range(288, _KP, _L):
            obits_v[pl.ds(t, _L)] = zero16 - 1
            oidx_v[pl.ds(t, _L)] = zero16

        def chunk(c, carry):
            c_sel, c_eq, idxv = carry
            v = bits_v[pl.ds(c * _L, _L)]
            gt = v > ustar
            eq = v == ustar
            eqi = eq.astype(jnp.int32)
            excl_eq = plsc.cumsum(eqi) - eqi
            sel = gt | (eq & ((c_eq + excl_eq) < need))
            plsc.store_compressed(obits_v.at[pl.ds(c_sel, _L)], v, mask=sel)
            plsc.store_compressed(oidx_v.at[pl.ds(c_sel, _L)], idxv, mask=sel)
            return (c_sel + jnp.sum(sel.astype(jnp.int32)),
                    c_eq + plsc.all_reduce_population_count(eq),
                    idxv + _L)

        lax.fori_loop(0, _NCHUNK, chunk,
                      (jnp.int32(0), zero16, iota16))

        for g in range(_KP // _L):
            gidx_v[pl.ds(g * _L, _L)] = oidx_v[pl.ds(g * _L, _L)] + base

        pltpu.async_copy(lab_hbm.at[gidx_v], olab_v, sem).wait()
        pltpu.async_copy(cx_hbm.at[gidx_v], ocx_v, sem).wait()
        pltpu.async_copy(cy_hbm.at[gidx_v], ocy_v, sem).wait()
        pltpu.async_copy(w_hbm.at[gidx_v], ow_v, sem).wait()
        pltpu.async_copy(h_hbm.at[gidx_v], oh_v, sem).wait()

        pltpu.sync_copy(obits_v.at[pl.ds(0, _KP)], obits_hbm.at[img])
        pltpu.sync_copy(oidx_v.at[pl.ds(0, _KP)], oidx_hbm.at[img])
        pltpu.sync_copy(olab_v, olab_hbm.at[img])
        pltpu.sync_copy(ocx_v, ocx_hbm.at[img])
        pltpu.sync_copy(ocy_v, ocy_hbm.at[img])
        pltpu.sync_copy(ow_v, ow_hbm.at[img])
        pltpu.sync_copy(oh_v, oh_hbm.at[img])


def _sc_compact_gather(bits, labels, cxp, cyp, wp, hp, meta):
    mesh = plsc.VectorSubcoreMesh(core_axis_name="c", subcore_axis_name="s")
    f32, i32 = jnp.float32, jnp.int32
    run = functools.partial(
        pl.kernel, mesh=mesh,
        compiler_params=pltpu.CompilerParams(needs_layout_passes=False),
        out_type=[
            jax.ShapeDtypeStruct((_B, _KP), i32),
            jax.ShapeDtypeStruct((_B, _KP), i32),
            jax.ShapeDtypeStruct((_B, _KP), i32),
            jax.ShapeDtypeStruct((_B, _KP), f32),
            jax.ShapeDtypeStruct((_B, _KP), f32),
            jax.ShapeDtypeStruct((_B, _KP), f32),
            jax.ShapeDtypeStruct((_B, _KP), f32),
        ],
        scratch_types=[
            pltpu.VMEM((_NQP,), i32),
            pltpu.VMEM((128,), i32),
            pltpu.VMEM((128,), i32),
            pltpu.VMEM((128,), i32),
            pltpu.VMEM((_KP + _L,), i32),
            pltpu.VMEM((_KP + _L,), i32),
            pltpu.VMEM((_KP,), i32),
            pltpu.VMEM((_KP,), i32),
            pltpu.VMEM((_KP,), f32),
            pltpu.VMEM((_KP,), f32),
            pltpu.VMEM((_KP,), f32),
            pltpu.VMEM((_KP,), f32),
            pltpu.SemaphoreType.DMA,
        ],
    )(_sc_body)
    base = jnp.broadcast_to(
        (jnp.arange(_B, dtype=i32) * _NQP)[:, None], (_B, 128))
    return run(bits, labels.reshape(-1), cxp.reshape(-1), cyp.reshape(-1),
               wp.reshape(-1), hp.reshape(-1),
               meta[0], meta[1], base)


def _order_kernel(bits_row_ref, bits_col_ref, idx_row_ref, idx_col_ref,
                  lab_ref, cx_ref, cy_ref, w_ref, h_ref, sizes_ref,
                  scores_out, labels_out, x1_out, y1_out, x2_out, y2_out):
    brow = bits_row_ref[0]
    bcol = bits_col_ref[0]
    irow = idx_row_ref[0]
    icol = idx_col_ref[0]

    before = (brow > bcol) | ((brow == bcol) & (irow < icol))
    rank = jnp.sum(before.astype(jnp.int32), axis=1, keepdims=True)

    outpos = lax.broadcasted_iota(jnp.int32, (_KP, _KP), 1)
    perm = (rank == outpos).astype(jnp.float32)

    srow = jnp.where(brow >= 0,
                     lax.bitcast_convert_type(brow, jnp.float32), 0.0)
    lrow = lab_ref[0].astype(jnp.float32)
    stream_i = lax.broadcasted_iota(jnp.int32, (8, _KP), 0)
    vt = jnp.where(stream_i == 0, jnp.broadcast_to(srow, (8, _KP)), 0.0)
    vt = jnp.where(stream_i == 1, jnp.broadcast_to(lrow, (8, _KP)), vt)
    vt = jnp.where(stream_i == 2, jnp.broadcast_to(cx_ref[0], (8, _KP)), vt)
    vt = jnp.where(stream_i == 3, jnp.broadcast_to(cy_ref[0], (8, _KP)), vt)
    vt = jnp.where(stream_i == 4, jnp.broadcast_to(w_ref[0], (8, _KP)), vt)
    vt = jnp.where(stream_i == 5, jnp.broadcast_to(h_ref[0], (8, _KP)), vt)

    r = lax.dot_general(vt, perm, (((1,), (0,)), ((), ())),
                        precision=lax.Precision.HIGHEST,
                        preferred_element_type=jnp.float32)

    i = pl.program_id(0)
    hrow = sizes_ref[pl.ds(i, 1), 0:1]
    wrow = sizes_ref[pl.ds(i, 1), 1:2]
    img_h = jnp.broadcast_to(hrow, (1, _KP))
    img_w = jnp.broadcast_to(wrow, (1, _KP))
    cxa = r[2:3, :]
    cya = r[3:4, :]
    wa = r[4:5, :]
    ha = r[5:6, :]
    scores_out[0] = r[0:1, :]
    labels_out[0] = r[1:2, :].astype(jnp.int32)
    x1_out[0] = (cxa - 0.5 * wa) * img_w
    y1_out[0] = (cya - 0.5 * ha) * img_h
    x2_out[0] = (cxa + 0.5 * wa) * img_w
    y2_out[0] = (cya + 0.5 * ha) * img_h


def _stage_c(cbits, cidx, clab, ccx, ccy, cw, ch, target_sizes):
    row = lambda a: a.reshape(_B, 1, _KP)
    col = lambda a: a.reshape(_B, _KP, 1)
    rspec = pl.BlockSpec((1, 1, _KP), lambda i: (i, 0, 0))
    cspec = pl.BlockSpec((1, _KP, 1), lambda i: (i, 0, 0))
    outs = pl.pallas_call(
        _order_kernel,
        grid=(_B,),
        in_specs=[rspec, cspec, rspec, cspec,
                  rspec, rspec, rspec, rspec, rspec,
                  pl.BlockSpec((_B, 2), lambda i: (0, 0))],
        out_specs=[rspec] * 6,
        out_shape=[
            jax.ShapeDtypeStruct((_B, 1, _KP), jnp.float32),
            jax.ShapeDtypeStruct((_B, 1, _KP), jnp.int32),
            jax.ShapeDtypeStruct((_B, 1, _KP), jnp.float32),
            jax.ShapeDtypeStruct((_B, 1, _KP), jnp.float32),
            jax.ShapeDtypeStruct((_B, 1, _KP), jnp.float32),
            jax.ShapeDtypeStruct((_B, 1, _KP), jnp.float32),
        ],
    )(row(cbits), col(cbits), row(cidx), col(cidx),
      row(clab), row(ccx), row(ccy), row(cw), row(ch), target_sizes)
    return [o.reshape(_B, _KP)[:, :_K] for o in outs]


def kernel(pred_logits, pred_boxes, target_sizes):
    logits_t = jnp.transpose(pred_logits, (0, 2, 1))
    logits_tp = logits_t
    pad_q = lambda a: jnp.pad(a, ((0, 0), (0, _NQP - _NQ)))
    cxp = pad_q(pred_boxes[..., 0])
    cyp = pad_q(pred_boxes[..., 1])
    wp = pad_q(pred_boxes[..., 2])
    hp = pad_q(pred_boxes[..., 3])

    bits, labels, ustar_arr, need_arr = _stage_a(logits_tp)
    comp = _sc_compact_gather(bits, labels, cxp, cyp, wp, hp,
                              (ustar_arr, need_arr))
    top_scores, top_labels, x1, y1, x2, y2 = _stage_c(*comp, target_sizes)
    top_boxes = jnp.stack([x1, y1, x2, y2], axis=-1)
    return top_scores, top_labels, top_boxes
